# baseline (speedup 1.0000x reference)
import jax
import jax.numpy as jnp
from jax.experimental import pallas as pl
from jax.experimental.pallas import tpu as pltpu

_B, _M, _C = 2, 4096, 16
_W = _C + _M
_R = 256
_G = 512


def _loss_body(scores_ref, lt_ref, clen_ref, cid_rows_ref, cid_pad_ref,
               owner_ref, out_ref, oh_ref, csize_ref):
    b = pl.program_id(0)
    t = pl.program_id(1)

    @pl.when(t == 0)
    def _build():
        cid_col = cid_pad_ref[0]
        gid = jax.lax.broadcasted_iota(jnp.int32, (_W, _G + 1), 1) - 1
        hit = jnp.logical_and(gid == cid_col.reshape(_W, 1), gid >= 0)
        onehot = jnp.where(hit, 1.0, 0.0)
        ones_col = jnp.where(gid == -1, 1.0, 0.0)
        oh_ref[...] = (onehot + ones_col).astype(jnp.bfloat16)
        csize_ref[...] = jnp.sum(onehot, axis=0, keepdims=True)

    s = scores_ref[0]
    lt = lt_ref[0]
    clen = clen_ref[0]
    cid_r = cid_rows_ref[0]
    owner = owner_ref[0]

    m = jnp.max(s, axis=1, keepdims=True)
    e = jnp.exp(s - m)

    row_g = t * _R + jax.lax.broadcasted_iota(jnp.int32, (_R, 1), 0)
    eye = owner == row_g
    e_diag = jnp.sum(jnp.where(eye, e, 0.0), axis=1, keepdims=True)
    e_bf = jnp.where(eye, 0.0, e).astype(jnp.bfloat16)

    A = jax.lax.dot_general(
        e_bf, oh_ref[...],
        dimension_numbers=(((1,), (0,)), ((), ())),
        preferred_element_type=jnp.float32,
    )
    sum_e = A[:, 0:1] + e_diag

    gr = jax.lax.broadcasted_iota(jnp.int32, (_R, _G + 1), 1) - 1
    row_oh = jnp.logical_and(gr == cid_r, gr >= 0)
    sum_mates_e = jnp.sum(jnp.where(row_oh, A, 0.0), axis=1, keepdims=True)
    cnt_same = jnp.sum(jnp.where(row_oh, csize_ref[...], 0.0), axis=1, keepdims=True)

    c16 = jax.lax.broadcasted_iota(jnp.int32, (_R, _C), 1)
    e_l = e[:, :_C]
    e_l_bf = e_bf[:, :_C].astype(jnp.float32)
    link_valid = c16 < clen
    sum_inv_l = jnp.sum(jnp.where(link_valid, 0.0, e_l_bf), axis=1, keepdims=True)
    gold_l = jnp.logical_and(lt != 0, link_valid)
    sum_gold_l = jnp.sum(jnp.where(gold_l, e_l, 0.0), axis=1, keepdims=True)
    cnt_gold_l = jnp.sum(jnp.where(gold_l, 1.0, 0.0), axis=1, keepdims=True)

    num_found = (cnt_same - 1.0) + cnt_gold_l
    self_f = jnp.where(num_found == 0.0, 1.0, 0.0)

    sum_all = sum_e - sum_inv_l
    sum_gold = sum_mates_e + self_f * e_diag + sum_gold_l

    contrib = jnp.sum(jnp.log(sum_all) - jnp.log(sum_gold), axis=0, keepdims=True)

    @pl.when(jnp.logical_and(b == 0, t == 0))
    def _init():
        out_ref[...] = jnp.zeros((1, 1), jnp.float32)

    out_ref[...] += contrib


@jax.jit
def kernel(scores, linker_targets, candidate_lengths, cluster_ids):
    B, M, W = scores.shape
    C = W - M
    clen = candidate_lengths.reshape(B, M, 1)
    cid_r = cluster_ids.reshape(B, M, 1)
    cid_p = jnp.concatenate(
        [jnp.full((B, 1, C), -1, jnp.int32), cluster_ids.reshape(B, 1, M)],
        axis=-1,
    )
    owner = jnp.concatenate(
        [jnp.full((1, 1, C), -1, jnp.int32),
         jnp.arange(M, dtype=jnp.int32).reshape(1, 1, M)],
        axis=-1,
    )

    grid = (B, M // _R)
    out = pl.pallas_call(
        _loss_body,
        grid=grid,
        in_specs=[
            pl.BlockSpec((1, _R, W), lambda b, t: (b, t, 0)),
            pl.BlockSpec((1, _R, C), lambda b, t: (b, t, 0)),
            pl.BlockSpec((1, _R, 1), lambda b, t: (b, t, 0)),
            pl.BlockSpec((1, _R, 1), lambda b, t: (b, t, 0)),
            pl.BlockSpec((1, 1, W), lambda b, t: (b, 0, 0)),
            pl.BlockSpec((1, 1, W), lambda b, t: (0, 0, 0)),
        ],
        out_specs=pl.BlockSpec((1, 1), lambda b, t: (0, 0)),
        out_shape=jax.ShapeDtypeStruct((1, 1), jnp.float32),
        scratch_shapes=[
            pltpu.VMEM((_W, _G + 1), jnp.bfloat16),
            pltpu.VMEM((1, _G + 1), jnp.float32),
        ],
        compiler_params=pltpu.CompilerParams(
            dimension_semantics=("arbitrary", "arbitrary"),
        ),
    )(scores, linker_targets, clen, cid_r, cid_p, owner)
    return out[0, 0]


# speedup vs baseline: 1.1128x; 1.1128x over previous
import jax
import jax.numpy as jnp
from jax.experimental import pallas as pl
from jax.experimental.pallas import tpu as pltpu

_B, _M, _C = 2, 4096, 16
_W = _C + _M
_R = 256
_G = 512
_NT = _M // _R


def _loss_body(scores_ref, lt_ref, clen_ref, cid_rows_ref, cid_pad_ref,
               out_ref, csize_ref, sdiag_ref):
    b = pl.program_id(0)
    t = pl.program_id(1)

    @pl.when(t == 0)
    def _build():
        cid_col = cid_pad_ref[0]
        gid = jax.lax.broadcasted_iota(jnp.int32, (_W, _G), 1)
        hit = gid == cid_col.reshape(_W, 1)
        csize_ref[...] = jnp.sum(jnp.where(hit, 1.0, 0.0), axis=0, keepdims=True)

    rr = jax.lax.broadcasted_iota(jnp.int32, (_R, _R), 0)
    cc = jax.lax.broadcasted_iota(jnp.int32, (_R, _R), 1)
    eye_rr = rr == cc
    for k in range(_NT):
        @pl.when(t == k)
        def _extract(k=k):
            win = scores_ref[0, :, (_C + k * _R):(_C + (k + 1) * _R)]
            sdiag_ref[...] = jnp.sum(jnp.where(eye_rr, win, 0.0), axis=1,
                                     keepdims=True)

    s = scores_ref[0]
    lt = lt_ref[0]
    clen = clen_ref[0]
    cid_r = cid_rows_ref[0]
    cid_p = cid_pad_ref[0]

    m = jnp.max(s, axis=1, keepdims=True)
    e = jnp.exp(s - m)
    sum_e = jnp.sum(e, axis=1, keepdims=True)
    same = cid_p == cid_r
    sum_same_e = jnp.sum(jnp.where(same, e, 0.0), axis=1, keepdims=True)

    e_diag = jnp.exp(sdiag_ref[...] - m)
    sum_mates_e = jnp.maximum(sum_same_e - e_diag, 0.0)

    gr = jax.lax.broadcasted_iota(jnp.int32, (_R, _G), 1)
    row_oh = gr == cid_r
    cnt_same = jnp.sum(jnp.where(row_oh, csize_ref[...], 0.0), axis=1,
                       keepdims=True)

    c16 = jax.lax.broadcasted_iota(jnp.int32, (_R, _C), 1)
    e_l = e[:, :_C]
    link_valid = c16 < clen
    sum_inv_l = jnp.sum(jnp.where(link_valid, 0.0, e_l), axis=1, keepdims=True)
    gold_l = jnp.logical_and(lt != 0, link_valid)
    sum_gold_l = jnp.sum(jnp.where(gold_l, e_l, 0.0), axis=1, keepdims=True)
    cnt_gold_l = jnp.sum(jnp.where(gold_l, 1.0, 0.0), axis=1, keepdims=True)

    num_found = (cnt_same - 1.0) + cnt_gold_l
    self_f = jnp.where(num_found == 0.0, 1.0, 0.0)

    sum_all = sum_e - sum_inv_l
    sum_gold = sum_mates_e + self_f * e_diag + sum_gold_l

    contrib = jnp.sum(jnp.log(sum_all) - jnp.log(sum_gold), axis=0, keepdims=True)

    @pl.when(jnp.logical_and(b == 0, t == 0))
    def _init():
        out_ref[...] = jnp.zeros((1, 1), jnp.float32)

    out_ref[...] += contrib


@jax.jit
def kernel(scores, linker_targets, candidate_lengths, cluster_ids):
    B, M, W = scores.shape
    C = W - M
    clen = candidate_lengths.reshape(B, M, 1)
    cid_r = cluster_ids.reshape(B, M, 1)
    cid_p = jnp.concatenate(
        [jnp.full((B, 1, C), -1, jnp.int32), cluster_ids.reshape(B, 1, M)],
        axis=-1,
    )

    grid = (B, M // _R)
    out = pl.pallas_call(
        _loss_body,
        grid=grid,
        in_specs=[
            pl.BlockSpec((1, _R, W), lambda b, t: (b, t, 0)),
            pl.BlockSpec((1, _R, C), lambda b, t: (b, t, 0)),
            pl.BlockSpec((1, _R, 1), lambda b, t: (b, t, 0)),
            pl.BlockSpec((1, _R, 1), lambda b, t: (b, t, 0)),
            pl.BlockSpec((1, 1, W), lambda b, t: (b, 0, 0)),
        ],
        out_specs=pl.BlockSpec((1, 1), lambda b, t: (0, 0)),
        out_shape=jax.ShapeDtypeStruct((1, 1), jnp.float32),
        scratch_shapes=[
            pltpu.VMEM((1, _G), jnp.float32),
            pltpu.VMEM((_R, 1), jnp.float32),
        ],
        compiler_params=pltpu.CompilerParams(
            dimension_semantics=("arbitrary", "arbitrary"),
        ),
    )(scores, linker_targets, clen, cid_r, cid_p)
    return out[0, 0]
